# baseline (speedup 1.0000x reference)
import functools
import jax
import jax.numpy as jnp
from jax.experimental import pallas as pl

_HID = 256
_OUT = 128
_NE = 8
_NL = 4
_FF = 512
_DIN = 768
_B = 4096
_NEG_INF = float("-inf")
_POS_INF = float("inf")


def _rup(n, m):
    return -(-n // m) * m


def _pick_bm(m):
    for bm in (512, 256, 400, 128, 64, 32, 16, 8, 4, 2, 1):
        if m % bm == 0:
            return bm
    return m


def _mm_body(x_ref, w_ref, b_ref, o_ref, *, act):
    y = jnp.dot(x_ref[...], w_ref[...], preferred_element_type=jnp.float32)
    y = y + b_ref[...]
    if act == "relu":
        y = jnp.maximum(y, 0.0)
    o_ref[...] = y


def _mm(x, w, b=None, act="none"):
    m, k = x.shape
    n = w.shape[1]
    if b is None:
        b = jnp.zeros((n,), jnp.float32)
    kp = _rup(k, 128)
    np_ = _rup(n, 128)
    if kp != k:
        x = jnp.pad(x, ((0, 0), (0, kp - k)))
        w = jnp.pad(w, ((0, kp - k), (0, 0)))
    if np_ != n:
        w = jnp.pad(w, ((0, 0), (0, np_ - n)))
        b = jnp.pad(b, (0, np_ - n))
    bm = _pick_bm(m)
    out = pl.pallas_call(
        functools.partial(_mm_body, act=act),
        grid=(m // bm,),
        in_specs=[
            pl.BlockSpec((bm, kp), lambda i: (i, 0)),
            pl.BlockSpec((kp, np_), lambda i: (0, 0)),
            pl.BlockSpec((1, np_), lambda i: (0, 0)),
        ],
        out_specs=pl.BlockSpec((bm, np_), lambda i: (i, 0)),
        out_shape=jax.ShapeDtypeStruct((m, np_), jnp.float32),
    )(x, w, b.reshape(1, np_))
    return out[:, :n] if np_ != n else out


def _comb_body(a_ref, r_ref, o_ref, *, final):
    v = a_ref[...] + r_ref[...]
    v = jnp.where(v > 0, v, jnp.exp(jnp.minimum(v, 0.0)) - 1.0)
    if final:
        nrm = jnp.sqrt(jnp.sum(v * v, axis=1, keepdims=True))
        v = v / jnp.maximum(nrm, 1e-12)
    o_ref[...] = v


def _combine(agg, res, final):
    m, n = agg.shape
    bm = _pick_bm(m)
    return pl.pallas_call(
        functools.partial(_comb_body, final=final),
        grid=(m // bm,),
        in_specs=[
            pl.BlockSpec((bm, n), lambda i: (i, 0)),
            pl.BlockSpec((bm, n), lambda i: (i, 0)),
        ],
        out_specs=pl.BlockSpec((bm, n), lambda i: (i, 0)),
        out_shape=jax.ShapeDtypeStruct((m, n), jnp.float32),
    )(agg, res)


def _moe_body(x_ref, nz_ref, gate_ref, nw_ref, w1_ref, b1_ref, w2_ref,
              b2_ref, o_ref, cs_ref):
    x = x_ref[...]
    g = jnp.dot(x, gate_ref[...], preferred_element_type=jnp.float32)
    nv = jnp.dot(x, nw_ref[...], preferred_element_type=jnp.float32)
    sp = jnp.maximum(nv, 0.0) + jnp.log1p(jnp.exp(-jnp.abs(nv)))
    lane = jax.lax.broadcasted_iota(jnp.int32, (1, 128), 1)
    h = g + nz_ref[...] * sp[:, 0:1]
    hb = jnp.where(lane < _NE, h, _POS_INF)
    mn1 = jnp.min(hb, axis=1, keepdims=True)
    j1 = jnp.min(jnp.where(hb == mn1, lane, 999), axis=1, keepdims=True)
    hb2 = jnp.where(lane == j1, _POS_INF, hb)
    mn2 = jnp.min(hb2, axis=1, keepdims=True)
    j2 = jnp.min(jnp.where(hb2 == mn2, lane, 999), axis=1, keepdims=True)
    hm = jnp.where((lane == j1) | (lane == j2) | (lane >= _NE),
                   _NEG_INF, h)
    m1 = jnp.max(hm, axis=1, keepdims=True)
    e = jnp.exp(hm - m1)
    lw = e / jnp.sum(e, axis=1, keepdims=True)
    cs_ref[...] = jnp.broadcast_to(
        jnp.sum(lw, axis=0, keepdims=True)[None], cs_ref.shape)
    i1 = jnp.min(jnp.where(hm == m1, lane, 999), axis=1, keepdims=True)
    hm2 = jnp.where(lane == i1, _NEG_INF, hm)
    m2 = jnp.max(hm2, axis=1, keepdims=True)
    i2 = jnp.min(jnp.where(hm2 == m2, lane, 999), axis=1, keepdims=True)
    lw = jnp.where((lane == i1) | (lane == i2), lw, 0.0)
    acc = jnp.zeros(x.shape, jnp.float32)
    for ei in range(_NE):
        y = jax.lax.dot_general(
            x, w1_ref[ei], (((1,), (1,)), ((), ())),
            preferred_element_type=jnp.float32) + b1_ref[ei]
        y = jnp.maximum(y, 0.0)
        z = jax.lax.dot_general(
            y, w2_ref[ei], (((1,), (1,)), ((), ())),
            preferred_element_type=jnp.float32) + b2_ref[ei]
        acc = acc + lw[:, ei:ei + 1] * z
    o_ref[...] = acc


def _moe(x, p, nz):
    bm = 512
    grid = _B // bm
    gate_t = jnp.pad(p["gate"].T, ((0, 0), (0, 128 - _NE)))
    nw_t = jnp.pad(p["noise"].T, ((0, 0), (0, 127)))
    b1 = p["e_b1"].reshape(_NE, 1, _FF)
    b2 = p["e_b2"].reshape(_NE, 1, _OUT)
    out, cs = pl.pallas_call(
        _moe_body,
        grid=(grid,),
        in_specs=[
            pl.BlockSpec((bm, _OUT), lambda i: (i, 0)),
            pl.BlockSpec((bm, 128), lambda i: (i, 0)),
            pl.BlockSpec((_OUT, 128), lambda i: (0, 0)),
            pl.BlockSpec((_OUT, 128), lambda i: (0, 0)),
            pl.BlockSpec((_NE, _FF, _OUT), lambda i: (0, 0, 0)),
            pl.BlockSpec((_NE, 1, _FF), lambda i: (0, 0, 0)),
            pl.BlockSpec((_NE, _OUT, _FF), lambda i: (0, 0, 0)),
            pl.BlockSpec((_NE, 1, _OUT), lambda i: (0, 0, 0)),
        ],
        out_specs=[
            pl.BlockSpec((bm, _OUT), lambda i: (i, 0)),
            pl.BlockSpec((1, 8, 128), lambda i: (i, 0, 0)),
        ],
        out_shape=[
            jax.ShapeDtypeStruct((_B, _OUT), jnp.float32),
            jax.ShapeDtypeStruct((grid, 8, 128), jnp.float32),
        ],
    )(x, nz, gate_t, nw_t, p["e_W1"], b1, p["e_W2"], b2)
    return out, jnp.sum(cs[:, 0, :], axis=0)[:_NE]


def _ln_in_kernel(x, g, b):
    m = jnp.mean(x, axis=1, keepdims=True)
    d = x - m
    v = jnp.mean(d * d, axis=1, keepdims=True)
    return d * jax.lax.rsqrt(v + 1e-5) * g + b


def _enc_body(z0_ref, z1_ref, z2_ref, wi_ref, bi_ref, wo_ref, bo_ref,
              g1_ref, e1_ref, f1_ref, c1_ref, f2_ref, c2_ref, g2_ref,
              e2_ref, hm_ref, ht_ref, cw_ref, cb_ref, o_ref):
    z = [z0_ref[...], z1_ref[...], z2_ref[...]]
    hmat = hm_ref[...]
    htm = ht_ref[...]
    for l in range(_NL):
        q, k, v = [], [], []
        for i in range(3):
            qkv = jax.lax.dot_general(
                z[i], wi_ref[l], (((1,), (1,)), ((), ())),
                preferred_element_type=jnp.float32) + bi_ref[l]
            q.append(qkv[:, :_OUT])
            k.append(qkv[:, _OUT:2 * _OUT])
            v.append(qkv[:, 2 * _OUT:3 * _OUT])
        att = []
        for i in range(3):
            s = [jnp.dot(q[i] * k[j], hmat,
                         preferred_element_type=jnp.float32)
                 for j in range(3)]
            mx = jnp.maximum(jnp.maximum(s[0], s[1]), s[2])
            e = [jnp.exp(sj - mx) for sj in s]
            den = e[0] + e[1] + e[2]
            o = jnp.zeros(z[i].shape, jnp.float32)
            for j in range(3):
                o = o + jnp.dot(e[j] / den, htm,
                                preferred_element_type=jnp.float32) * v[j]
            att.append(jax.lax.dot_general(
                o, wo_ref[l], (((1,), (1,)), ((), ())),
                preferred_element_type=jnp.float32) + bo_ref[l])
        z = [_ln_in_kernel(z[i] + att[i], g1_ref[l], e1_ref[l])
             for i in range(3)]
        ff = []
        for i in range(3):
            y = jax.lax.dot_general(
                z[i], f1_ref[l], (((1,), (1,)), ((), ())),
                preferred_element_type=jnp.float32) + c1_ref[l]
            y = jnp.maximum(y, 0.0)
            ff.append(jax.lax.dot_general(
                y, f2_ref[l], (((1,), (1,)), ((), ())),
                preferred_element_type=jnp.float32) + c2_ref[l])
        z = [_ln_in_kernel(z[i] + ff[i], g2_ref[l], e2_ref[l])
             for i in range(3)]
    flat = jnp.concatenate(z, axis=1)
    o_ref[...] = jax.lax.dot_general(
        flat, cw_ref[...], (((1,), (1,)), ((), ())),
        preferred_element_type=jnp.float32) + cb_ref[...]


def _encoder(z0, z1, z2, enc, cls_w, cls_b):
    hd = 32
    lane = jnp.arange(128)
    head = jnp.arange(128) // hd
    hmat = jnp.where((head[:, None] == lane[None, :]) & (lane[None, :] < 4),
                     1.0 / jnp.sqrt(jnp.float32(hd)), 0.0).astype(jnp.float32)
    htm = jnp.where((lane[:, None] < 4) & (head[None, :] == lane[:, None]),
                    1.0, 0.0).astype(jnp.float32)
    st = lambda nm: jnp.stack([lp[nm] for lp in enc])
    wi, wo = st("Wi"), st("Wo")
    bi = st("bi").reshape(_NL, 1, 3 * _OUT)
    bo = st("bo").reshape(_NL, 1, _OUT)
    g1 = st("ln1_g").reshape(_NL, 1, _OUT)
    e1 = st("ln1_b").reshape(_NL, 1, _OUT)
    f1, f2 = st("ff_W1"), st("ff_W2")
    c1 = st("ff_b1").reshape(_NL, 1, _FF)
    c2 = st("ff_b2").reshape(_NL, 1, _OUT)
    g2 = st("ln2_g").reshape(_NL, 1, _OUT)
    e2 = st("ln2_b").reshape(_NL, 1, _OUT)
    cwp = jnp.pad(cls_w, ((0, 126), (0, 0)))
    cbp = jnp.pad(cls_b, (0, 126)).reshape(1, 128)
    bm = 512
    grid = _B // bm
    full = lambda shp: pl.BlockSpec(shp, lambda i: (0,) * len(shp))
    out = pl.pallas_call(
        _enc_body,
        grid=(grid,),
        in_specs=[
            pl.BlockSpec((bm, _OUT), lambda i: (i, 0)),
            pl.BlockSpec((bm, _OUT), lambda i: (i, 0)),
            pl.BlockSpec((bm, _OUT), lambda i: (i, 0)),
            full(wi.shape), full(bi.shape), full(wo.shape), full(bo.shape),
            full(g1.shape), full(e1.shape), full(f1.shape), full(c1.shape),
            full(f2.shape), full(c2.shape), full(g2.shape), full(e2.shape),
            full(hmat.shape), full(htm.shape), full(cwp.shape),
            full(cbp.shape),
        ],
        out_specs=pl.BlockSpec((bm, 128), lambda i: (i, 0)),
        out_shape=jax.ShapeDtypeStruct((_B, 128), jnp.float32),
    )(z0, z1, z2, wi, bi, wo, bo, g1, e1, f1, c1, f2, c2, g2, e2,
      hmat, htm, cwp, cbp)
    return out[:, :2]


def _hgn_layer(h, src, dst, etype, p, final):
    n = h.shape[0]
    wx = _mm(h, p["W"].T)
    a_dst = p["a"][0, :_HID]
    a_src = p["a"][0, _HID:2 * _HID]
    a_rel = p["a"][0, 2 * _HID:]
    ab = _mm(wx, jnp.stack([a_dst, a_src], axis=1))
    ra = (p["rel"] @ p["W_r"].T) @ a_rel
    alpha = ab[dst, 0] + ab[src, 1] + ra[etype]
    alpha = jnp.where(alpha >= 0, alpha, 0.2 * alpha)
    amax = jax.ops.segment_max(alpha, dst, num_segments=n)
    amax = jnp.where(jnp.isfinite(amax), amax, 0.0)
    e = jnp.exp(alpha - amax[dst])
    s = jax.ops.segment_sum(e, dst, num_segments=n)
    attn = e / (s[dst] + 1e-16)
    agg = jax.ops.segment_sum(wx[src] * attn[:, None], dst, num_segments=n)
    res = _mm(h, p["W_res"].T)
    return _combine(agg, res, final)


def kernel(x, edge_index, edge_type, batch_size, params):
    src, dst = edge_index[0], edge_index[1]
    start = batch_size - _B
    xb = jax.lax.dynamic_slice_in_dim(x, start, _B, axis=0)
    meta_f = xb[:, _DIN:]
    text = xb[:, :_DIN]

    h = _mm(x, params["lin1_W"].T, params["lin1_b"], act="relu")
    h = _hgn_layer(h, src, dst, edge_type, params["conv1"], False)
    h = _hgn_layer(h, src, dst, edge_type, params["conv2"], True)
    h = _mm(h, params["lin2_W"].T, params["lin2_b"], act="relu")
    x_g = jax.lax.dynamic_slice_in_dim(h, start, _B, axis=0)

    x_t = _mm(text, params["text_W"].T, params["text_b"], act="relu")
    x_m = _mm(meta_f, params["meta_W1"].T, params["meta_b1"], act="relu")
    x_m = _mm(x_m, params["meta_W2"].T, params["meta_b2"], act="relu")

    kroot = jax.random.key(123)
    outs = []
    losses = []
    for i, xi in enumerate((x_g, x_t, x_m)):
        nz = jax.random.normal(jax.random.fold_in(kroot, i), (_B, _NE),
                               dtype=jnp.float32)
        nz = jnp.pad(nz, ((0, 0), (0, 128 - _NE)))
        o, tmp = _moe(xi, params["moe"][i], nz)
        outs.append(o)
        losses.append((jnp.std(tmp, ddof=1) / jnp.mean(tmp)) ** 2)

    logits = _encoder(outs[0], outs[1], outs[2], params["enc"],
                      params["cls_W"], params["cls_b"])
    return logits, losses[0] + losses[1] + losses[2]


# speedup vs baseline: 3.4372x; 3.4372x over previous
import functools
import jax
import jax.numpy as jnp
from jax.experimental import pallas as pl

_HID = 256
_OUT = 128
_NE = 8
_NL = 4
_FF = 512
_DIN = 768
_B = 4096
_NEG_INF = float("-inf")
_POS_INF = float("inf")


def _rup(n, m):
    return -(-n // m) * m


def _pick_bm(m):
    for bm in (512, 256, 400, 128, 64, 32, 16, 8, 4, 2, 1):
        if m % bm == 0:
            return bm
    return m


def _mm_body(x_ref, w_ref, b_ref, o_ref, *, act):
    y = jnp.dot(x_ref[...], w_ref[...], preferred_element_type=jnp.float32)
    y = y + b_ref[...]
    if act == "relu":
        y = jnp.maximum(y, 0.0)
    o_ref[...] = y


def _mm(x, w, b=None, act="none"):
    m, k = x.shape
    n = w.shape[1]
    if b is None:
        b = jnp.zeros((n,), jnp.float32)
    kp = _rup(k, 128)
    np_ = _rup(n, 128)
    if kp != k:
        x = jnp.pad(x, ((0, 0), (0, kp - k)))
        w = jnp.pad(w, ((0, kp - k), (0, 0)))
    if np_ != n:
        w = jnp.pad(w, ((0, 0), (0, np_ - n)))
        b = jnp.pad(b, (0, np_ - n))
    bm = _pick_bm(m)
    out = pl.pallas_call(
        functools.partial(_mm_body, act=act),
        grid=(m // bm,),
        in_specs=[
            pl.BlockSpec((bm, kp), lambda i: (i, 0)),
            pl.BlockSpec((kp, np_), lambda i: (0, 0)),
            pl.BlockSpec((1, np_), lambda i: (0, 0)),
        ],
        out_specs=pl.BlockSpec((bm, np_), lambda i: (i, 0)),
        out_shape=jax.ShapeDtypeStruct((m, np_), jnp.float32),
    )(x, w, b.reshape(1, np_))
    return out[:, :n] if np_ != n else out


def _comb_body(a_ref, s_ref, r_ref, o_ref, *, final):
    v = a_ref[...] / (s_ref[...] + 1e-16) + r_ref[...]
    v = jnp.where(v > 0, v, jnp.exp(jnp.minimum(v, 0.0)) - 1.0)
    if final:
        nrm = jnp.sqrt(jnp.sum(v * v, axis=1, keepdims=True))
        v = v / jnp.maximum(nrm, 1e-12)
    o_ref[...] = v


def _combine(agg, s, res, final):
    m, n = agg.shape
    bm = _pick_bm(m)
    return pl.pallas_call(
        functools.partial(_comb_body, final=final),
        grid=(m // bm,),
        in_specs=[
            pl.BlockSpec((bm, n), lambda i: (i, 0)),
            pl.BlockSpec((bm, 1), lambda i: (i, 0)),
            pl.BlockSpec((bm, n), lambda i: (i, 0)),
        ],
        out_specs=pl.BlockSpec((bm, n), lambda i: (i, 0)),
        out_shape=jax.ShapeDtypeStruct((m, n), jnp.float32),
    )(agg, s.reshape(m, 1), res)


def _moe_body(x_ref, nz_ref, gate_ref, nw_ref, w1_ref, b1_ref, w2_ref,
              b2_ref, o_ref, cs_ref):
    x = x_ref[...]
    g = jnp.dot(x, gate_ref[...], preferred_element_type=jnp.float32)
    nv = jnp.dot(x, nw_ref[...], preferred_element_type=jnp.float32)
    sp = jnp.maximum(nv, 0.0) + jnp.log1p(jnp.exp(-jnp.abs(nv)))
    lane = jax.lax.broadcasted_iota(jnp.int32, (1, 128), 1)
    h = g + nz_ref[...] * sp[:, 0:1]
    hb = jnp.where(lane < _NE, h, _POS_INF)
    mn1 = jnp.min(hb, axis=1, keepdims=True)
    j1 = jnp.min(jnp.where(hb == mn1, lane, 999), axis=1, keepdims=True)
    hb2 = jnp.where(lane == j1, _POS_INF, hb)
    mn2 = jnp.min(hb2, axis=1, keepdims=True)
    j2 = jnp.min(jnp.where(hb2 == mn2, lane, 999), axis=1, keepdims=True)
    hm = jnp.where((lane == j1) | (lane == j2) | (lane >= _NE),
                   _NEG_INF, h)
    m1 = jnp.max(hm, axis=1, keepdims=True)
    e = jnp.exp(hm - m1)
    lw = e / jnp.sum(e, axis=1, keepdims=True)
    cs_ref[...] = jnp.broadcast_to(
        jnp.sum(lw, axis=0, keepdims=True)[None], cs_ref.shape)
    i1 = jnp.min(jnp.where(hm == m1, lane, 999), axis=1, keepdims=True)
    hm2 = jnp.where(lane == i1, _NEG_INF, hm)
    m2 = jnp.max(hm2, axis=1, keepdims=True)
    i2 = jnp.min(jnp.where(hm2 == m2, lane, 999), axis=1, keepdims=True)
    lw = jnp.where((lane == i1) | (lane == i2), lw, 0.0)
    acc = jnp.zeros(x.shape, jnp.float32)
    for ei in range(_NE):
        y = jax.lax.dot_general(
            x, w1_ref[ei], (((1,), (1,)), ((), ())),
            preferred_element_type=jnp.float32) + b1_ref[ei]
        y = jnp.maximum(y, 0.0)
        z = jax.lax.dot_general(
            y, w2_ref[ei], (((1,), (1,)), ((), ())),
            preferred_element_type=jnp.float32) + b2_ref[ei]
        acc = acc + lw[:, ei:ei + 1] * z
    o_ref[...] = acc


def _moe(x, p, nz):
    bm = 512
    grid = _B // bm
    gate_t = jnp.pad(p["gate"].T, ((0, 0), (0, 128 - _NE)))
    nw_t = jnp.pad(p["noise"].T, ((0, 0), (0, 127)))
    b1 = p["e_b1"].reshape(_NE, 1, _FF)
    b2 = p["e_b2"].reshape(_NE, 1, _OUT)
    out, cs = pl.pallas_call(
        _moe_body,
        grid=(grid,),
        in_specs=[
            pl.BlockSpec((bm, _OUT), lambda i: (i, 0)),
            pl.BlockSpec((bm, 128), lambda i: (i, 0)),
            pl.BlockSpec((_OUT, 128), lambda i: (0, 0)),
            pl.BlockSpec((_OUT, 128), lambda i: (0, 0)),
            pl.BlockSpec((_NE, _FF, _OUT), lambda i: (0, 0, 0)),
            pl.BlockSpec((_NE, 1, _FF), lambda i: (0, 0, 0)),
            pl.BlockSpec((_NE, _OUT, _FF), lambda i: (0, 0, 0)),
            pl.BlockSpec((_NE, 1, _OUT), lambda i: (0, 0, 0)),
        ],
        out_specs=[
            pl.BlockSpec((bm, _OUT), lambda i: (i, 0)),
            pl.BlockSpec((1, 8, 128), lambda i: (i, 0, 0)),
        ],
        out_shape=[
            jax.ShapeDtypeStruct((_B, _OUT), jnp.float32),
            jax.ShapeDtypeStruct((grid, 8, 128), jnp.float32),
        ],
    )(x, nz, gate_t, nw_t, p["e_W1"], b1, p["e_W2"], b2)
    return out, jnp.sum(cs[:, 0, :], axis=0)[:_NE]


def _ln_in_kernel(x, g, b):
    m = jnp.mean(x, axis=1, keepdims=True)
    d = x - m
    v = jnp.mean(d * d, axis=1, keepdims=True)
    return d * jax.lax.rsqrt(v + 1e-5) * g + b


def _enc_body(z0_ref, z1_ref, z2_ref, wi_ref, bi_ref, wo_ref, bo_ref,
              g1_ref, e1_ref, f1_ref, c1_ref, f2_ref, c2_ref, g2_ref,
              e2_ref, hm_ref, ht_ref, cw_ref, cb_ref, o_ref):
    z = [z0_ref[...], z1_ref[...], z2_ref[...]]
    hmat = hm_ref[...]
    htm = ht_ref[...]
    for l in range(_NL):
        q, k, v = [], [], []
        for i in range(3):
            qkv = jax.lax.dot_general(
                z[i], wi_ref[l], (((1,), (1,)), ((), ())),
                preferred_element_type=jnp.float32) + bi_ref[l]
            q.append(qkv[:, :_OUT])
            k.append(qkv[:, _OUT:2 * _OUT])
            v.append(qkv[:, 2 * _OUT:3 * _OUT])
        att = []
        for i in range(3):
            s = [jnp.dot(q[i] * k[j], hmat,
                         preferred_element_type=jnp.float32)
                 for j in range(3)]
            mx = jnp.maximum(jnp.maximum(s[0], s[1]), s[2])
            e = [jnp.exp(sj - mx) for sj in s]
            den = e[0] + e[1] + e[2]
            o = jnp.zeros(z[i].shape, jnp.float32)
            for j in range(3):
                o = o + jnp.dot(e[j] / den, htm,
                                preferred_element_type=jnp.float32) * v[j]
            att.append(jax.lax.dot_general(
                o, wo_ref[l], (((1,), (1,)), ((), ())),
                preferred_element_type=jnp.float32) + bo_ref[l])
        z = [_ln_in_kernel(z[i] + att[i], g1_ref[l], e1_ref[l])
             for i in range(3)]
        ff = []
        for i in range(3):
            y = jax.lax.dot_general(
                z[i], f1_ref[l], (((1,), (1,)), ((), ())),
                preferred_element_type=jnp.float32) + c1_ref[l]
            y = jnp.maximum(y, 0.0)
            ff.append(jax.lax.dot_general(
                y, f2_ref[l], (((1,), (1,)), ((), ())),
                preferred_element_type=jnp.float32) + c2_ref[l])
        z = [_ln_in_kernel(z[i] + ff[i], g2_ref[l], e2_ref[l])
             for i in range(3)]
    flat = jnp.concatenate(z, axis=1)
    o_ref[...] = jax.lax.dot_general(
        flat, cw_ref[...], (((1,), (1,)), ((), ())),
        preferred_element_type=jnp.float32) + cb_ref[...]


def _encoder(z0, z1, z2, enc, cls_w, cls_b):
    hd = 32
    lane = jnp.arange(128)
    head = jnp.arange(128) // hd
    hmat = jnp.where((head[:, None] == lane[None, :]) & (lane[None, :] < 4),
                     1.0 / jnp.sqrt(jnp.float32(hd)), 0.0).astype(jnp.float32)
    htm = jnp.where((lane[:, None] < 4) & (head[None, :] == lane[:, None]),
                    1.0, 0.0).astype(jnp.float32)
    st = lambda nm: jnp.stack([lp[nm] for lp in enc])
    wi, wo = st("Wi"), st("Wo")
    bi = st("bi").reshape(_NL, 1, 3 * _OUT)
    bo = st("bo").reshape(_NL, 1, _OUT)
    g1 = st("ln1_g").reshape(_NL, 1, _OUT)
    e1 = st("ln1_b").reshape(_NL, 1, _OUT)
    f1, f2 = st("ff_W1"), st("ff_W2")
    c1 = st("ff_b1").reshape(_NL, 1, _FF)
    c2 = st("ff_b2").reshape(_NL, 1, _OUT)
    g2 = st("ln2_g").reshape(_NL, 1, _OUT)
    e2 = st("ln2_b").reshape(_NL, 1, _OUT)
    cwp = jnp.pad(cls_w, ((0, 126), (0, 0)))
    cbp = jnp.pad(cls_b, (0, 126)).reshape(1, 128)
    bm = 512
    grid = _B // bm
    full = lambda shp: pl.BlockSpec(shp, lambda i: (0,) * len(shp))
    out = pl.pallas_call(
        _enc_body,
        grid=(grid,),
        in_specs=[
            pl.BlockSpec((bm, _OUT), lambda i: (i, 0)),
            pl.BlockSpec((bm, _OUT), lambda i: (i, 0)),
            pl.BlockSpec((bm, _OUT), lambda i: (i, 0)),
            full(wi.shape), full(bi.shape), full(wo.shape), full(bo.shape),
            full(g1.shape), full(e1.shape), full(f1.shape), full(c1.shape),
            full(f2.shape), full(c2.shape), full(g2.shape), full(e2.shape),
            full(hmat.shape), full(htm.shape), full(cwp.shape),
            full(cbp.shape),
        ],
        out_specs=pl.BlockSpec((bm, 128), lambda i: (i, 0)),
        out_shape=jax.ShapeDtypeStruct((_B, 128), jnp.float32),
    )(z0, z1, z2, wi, bi, wo, bo, g1, e1, f1, c1, f2, c2, g2, e2,
      hmat, htm, cwp, cbp)
    return out[:, :2]


def _hgn_layer(h, src, dst, etype, p, final):
    n = h.shape[0]
    wx = _mm(h, p["W"].T)
    a_dst = p["a"][0, :_HID]
    a_src = p["a"][0, _HID:2 * _HID]
    a_rel = p["a"][0, 2 * _HID:]
    ab = _mm(wx, jnp.stack([a_dst, a_src], axis=1))
    ra = (p["rel"] @ p["W_r"].T) @ a_rel
    g_d = jnp.take(ab, dst, axis=0)[:, 0]
    g_s = jnp.take(ab, src, axis=0)[:, 1]
    ra_sel = jnp.where(etype == 0, ra[0],
                       jnp.where(etype == 1, ra[1], ra[2]))
    alpha = g_d + g_s + ra_sel
    alpha = jnp.where(alpha >= 0, alpha, 0.2 * alpha)
    e = jnp.exp(alpha)
    s = jax.ops.segment_sum(e, dst, num_segments=n)
    agg = jax.ops.segment_sum(wx[src] * e[:, None], dst, num_segments=n)
    res = _mm(h, p["W_res"].T)
    return _combine(agg, s, res, final)


def kernel(x, edge_index, edge_type, batch_size, params):
    src, dst = edge_index[0], edge_index[1]
    start = batch_size - _B
    xb = jax.lax.dynamic_slice_in_dim(x, start, _B, axis=0)
    meta_f = xb[:, _DIN:]
    text = xb[:, :_DIN]

    h = _mm(x, params["lin1_W"].T, params["lin1_b"], act="relu")
    h = _hgn_layer(h, src, dst, edge_type, params["conv1"], False)
    h = _hgn_layer(h, src, dst, edge_type, params["conv2"], True)
    h = _mm(h, params["lin2_W"].T, params["lin2_b"], act="relu")
    x_g = jax.lax.dynamic_slice_in_dim(h, start, _B, axis=0)

    x_t = _mm(text, params["text_W"].T, params["text_b"], act="relu")
    x_m = _mm(meta_f, params["meta_W1"].T, params["meta_b1"], act="relu")
    x_m = _mm(x_m, params["meta_W2"].T, params["meta_b2"], act="relu")

    kroot = jax.random.key(123)
    outs = []
    losses = []
    for i, xi in enumerate((x_g, x_t, x_m)):
        nz = jax.random.normal(jax.random.fold_in(kroot, i), (_B, _NE),
                               dtype=jnp.float32)
        nz = jnp.pad(nz, ((0, 0), (0, 128 - _NE)))
        o, tmp = _moe(xi, params["moe"][i], nz)
        outs.append(o)
        losses.append((jnp.std(tmp, ddof=1) / jnp.mean(tmp)) ** 2)

    logits = _encoder(outs[0], outs[1], outs[2], params["enc"],
                      params["cls_W"], params["cls_b"])
    return logits, losses[0] + losses[1] + losses[2]
